# Initial kernel scaffold; baseline (speedup 1.0000x reference)
import jax
import jax.numpy as jnp
from jax.experimental import pallas as pl


def kernel(nodeState, edge_index, embed, Wq, bq, Wk, bk, Wv, bv, Wo, bo, g1, be1, W1, b1, W2, b2, g2, be2, Wc, bc):
    raise NotImplementedError("write your pallas kernel here")


# speedup vs baseline: 31.6324x; 31.6324x over previous
import functools

import jax
import jax.numpy as jnp
import numpy as np
from jax import lax
from jax.experimental import pallas as pl
from jax.experimental.pallas import tpu as pltpu
from jax.experimental.pallas import tpu_sc as plsc

_N = 10000
_E = 320000
_D = 128
_H = 8
_DH = 16
_SCALE = float(np.sqrt(_DH))

_NC = 2
_NS = 16
_NW = _NC * _NS
_CH = 128
_W144 = _D + 16

_mesh = plsc.VectorSubcoreMesh(core_axis_name="c", subcore_axis_name="s")


def _sc_gather_rows(table, idx):
    B = idx.shape[0]
    Dd = table.shape[1]
    nchunks = B // _CH
    per_w = -(-nchunks // _NW)

    @functools.partial(
        pl.kernel,
        out_type=jax.ShapeDtypeStruct((B, Dd), jnp.float32),
        mesh=_mesh,
        scratch_types=[
            pltpu.VMEM((_CH,), jnp.int32),
            pltpu.VMEM((_CH, Dd), jnp.float32),
            pltpu.SemaphoreType.DMA,
        ],
    )
    def k(table_h, idx_h, out_h, iv, buf, sem):
        w = lax.axis_index("s") * _NC + lax.axis_index("c")

        def step(kk, carry):
            ci = w + _NW * kk

            @pl.when(ci < nchunks)
            def _():
                base = ci * _CH
                pltpu.sync_copy(idx_h.at[pl.ds(base, _CH)], iv)
                pltpu.async_copy(table_h.at[iv], buf, sem).wait()
                pltpu.sync_copy(buf, out_h.at[pl.ds(base, _CH)])

            return carry

        lax.fori_loop(0, per_w, step, 0)

    return k(table, idx)


def _sc_gather_edges(kh, qh, vh, src, dst):
    nchunks = _E // _CH
    per_w = -(-nchunks // _NW)

    @functools.partial(
        pl.kernel,
        out_type=[jax.ShapeDtypeStruct((_E, _D), jnp.float32)] * 3,
        mesh=_mesh,
        scratch_types=[
            pltpu.VMEM((_CH,), jnp.int32),
            pltpu.VMEM((_CH,), jnp.int32),
            pltpu.VMEM((_CH, _D), jnp.float32),
            pltpu.VMEM((_CH, _D), jnp.float32),
            pltpu.VMEM((_CH, _D), jnp.float32),
            pltpu.SemaphoreType.DMA,
            pltpu.SemaphoreType.DMA,
            pltpu.SemaphoreType.DMA,
        ],
    )
    def k(kh_h, qh_h, vh_h, src_h, dst_h, ko, qo, vo,
          siv, div, kbuf, qbuf, vbuf, semk, semq, semv):
        w = lax.axis_index("s") * _NC + lax.axis_index("c")

        def step(kk, carry):
            ci = w + _NW * kk

            @pl.when(ci < nchunks)
            def _():
                base = ci * _CH
                pltpu.sync_copy(src_h.at[pl.ds(base, _CH)], siv)
                pltpu.sync_copy(dst_h.at[pl.ds(base, _CH)], div)
                dk = pltpu.async_copy(kh_h.at[siv], kbuf, semk)
                dq = pltpu.async_copy(qh_h.at[div], qbuf, semq)
                dv = pltpu.async_copy(vh_h.at[siv], vbuf, semv)
                dk.wait()
                dq.wait()
                dv.wait()
                pltpu.sync_copy(kbuf, ko.at[pl.ds(base, _CH)])
                pltpu.sync_copy(qbuf, qo.at[pl.ds(base, _CH)])
                pltpu.sync_copy(vbuf, vo.at[pl.ds(base, _CH)])

            return carry

        lax.fori_loop(0, per_w, step, 0)

    return k(kh, qh, vh, src, dst)


def _sc_scatter_add(wrows, dst):
    nchunks = _E // _CH
    per_w = -(-nchunks // _NW)
    zr = 80
    nzc = _N // zr
    per_s = -(-nzc // _NS)

    @functools.partial(
        pl.kernel,
        out_type=jax.ShapeDtypeStruct((_NC, _N, _W144), jnp.float32),
        mesh=_mesh,
        compiler_params=pltpu.CompilerParams(use_tc_tiling_on_sc=False),
        scratch_types=[
            pltpu.VMEM_SHARED((_N, _W144), jnp.float32),
            pltpu.VMEM((_CH,), jnp.int32),
            pltpu.VMEM((_CH, _W144), jnp.float32),
            pltpu.VMEM((zr, _W144), jnp.float32),
            pltpu.SemaphoreType.DMA,
        ],
    )
    def k(w_h, dst_h, out_h, acc, iv, buf, zbuf, sem):
        c = lax.axis_index("c")
        s = lax.axis_index("s")
        w = s * _NC + c

        def zrow(i, carry):
            for j in range(_W144 // 16):
                zbuf[i, pl.ds(j * 16, 16)] = jnp.zeros((16,), jnp.float32)
            return carry

        lax.fori_loop(0, zr, zrow, 0)

        def zchunk(t, carry):
            j = s + _NS * t

            @pl.when(j < nzc)
            def _():
                pltpu.sync_copy(zbuf, acc.at[pl.ds(j * zr, zr)])

            return carry

        lax.fori_loop(0, per_s, zchunk, 0)
        plsc.subcore_barrier()

        def step(kk, carry):
            ci = w + _NW * kk

            @pl.when(ci < nchunks)
            def _():
                base = ci * _CH
                pltpu.sync_copy(dst_h.at[pl.ds(base, _CH)], iv)
                pltpu.sync_copy(w_h.at[pl.ds(base, _CH)], buf)
                pltpu.sync_copy(buf, acc.at[iv], add=True)

            return carry

        lax.fori_loop(0, per_w, step, 0)
        plsc.subcore_barrier()

        def dchunk(t, carry):
            j = s + _NS * t

            @pl.when(j < nzc)
            def _():
                pltpu.sync_copy(acc.at[pl.ds(j * zr, zr)],
                                out_h.at[c, pl.ds(j * zr, zr)])

            return carry

        lax.fori_loop(0, per_s, dchunk, 0)

    return k(wrows, dst)


def _tc_qkv(h, wq, bq, wk, bk, wv, bv):
    def body(h_r, wq_r, bq_r, wk_r, bk_r, wv_r, bv_r, qo, ko, vo):
        hh = h_r[...]
        qo[...] = jnp.dot(hh, wq_r[...], preferred_element_type=jnp.float32) + bq_r[...]
        ko[...] = jnp.dot(hh, wk_r[...], preferred_element_type=jnp.float32) + bk_r[...]
        vo[...] = jnp.dot(hh, wv_r[...], preferred_element_type=jnp.float32) + bv_r[...]

    return pl.pallas_call(
        body,
        out_shape=[jax.ShapeDtypeStruct((_N, _D), jnp.float32)] * 3,
    )(h, wq, bq, wk, bk, wv, bv)


def _tc_edge_weight(ke, qe, ve, sel_a, sel_b, sel_p):
    R = 4000
    grid = _E // R

    def body(ks, qs, vs, sa, sb, sp, wo):
        prod = ks[...] * qs[...]
        sc = jnp.dot(prod, sa[...], preferred_element_type=jnp.float32) * (1.0 / _SCALE)
        e = jnp.exp(jnp.clip(sc, -5.0, 5.0))
        wv = vs[...] * jnp.dot(e, sb[...], preferred_element_type=jnp.float32)
        ep = jnp.dot(e, sp[...], preferred_element_type=jnp.float32)
        wo[...] = jnp.concatenate([wv, ep], axis=1)

    return pl.pallas_call(
        body,
        grid=(grid,),
        in_specs=[
            pl.BlockSpec((R, _D), lambda i: (i, 0)),
            pl.BlockSpec((R, _D), lambda i: (i, 0)),
            pl.BlockSpec((R, _D), lambda i: (i, 0)),
            pl.BlockSpec((_D, _H), lambda i: (0, 0)),
            pl.BlockSpec((_H, _D), lambda i: (0, 0)),
            pl.BlockSpec((_H, 16), lambda i: (0, 0)),
        ],
        out_specs=pl.BlockSpec((R, _W144), lambda i: (i, 0)),
        out_shape=jax.ShapeDtypeStruct((_E, _W144), jnp.float32),
    )(ke, qe, ve, sel_a, sel_b, sel_p)


def _tc_post(h, parts, sel_b, wo, bo, g1, be1, w1, b1, w2, b2, g2, be2):
    def body(h_r, p_r, sb, wo_r, bo_r, g1_r, be1_r, w1_r, b1_r, w2_r, b2_r,
             g2_r, be2_r, ho):
        p0 = p_r[0]
        p1 = p_r[1]
        wv = p0[:, :_D] + p1[:, :_D]
        z8 = p0[:, _D:_D + _H] + p1[:, _D:_D + _H]
        zb = jnp.dot(z8, sb[...], preferred_element_type=jnp.float32)
        att = wv / (zb + 1e-6)
        head = jnp.dot(att, wo_r[...], preferred_element_type=jnp.float32) + bo_r[...]
        x = h_r[...] + head
        mu = jnp.mean(x, axis=0, keepdims=True)
        var = jnp.mean((x - mu) ** 2, axis=0, keepdims=True)
        xn = g1_r[...] * (x - mu) * lax.rsqrt(var + 1e-5) + be1_r[...]
        ff = jnp.maximum(
            jnp.dot(xn, w1_r[...], preferred_element_type=jnp.float32) + b1_r[...], 0.0)
        ff = jnp.dot(ff, w2_r[...], preferred_element_type=jnp.float32) + b2_r[...]
        y = xn + ff
        mu2 = jnp.mean(y, axis=0, keepdims=True)
        var2 = jnp.mean((y - mu2) ** 2, axis=0, keepdims=True)
        ho[...] = g2_r[...] * (y - mu2) * lax.rsqrt(var2 + 1e-5) + be2_r[...]

    return pl.pallas_call(
        body,
        out_shape=jax.ShapeDtypeStruct((_N, _D), jnp.float32),
    )(h, parts, sel_b, wo, bo, g1, be1, w1, b1, w2, b2, g2, be2)


def _tc_final(h, wc, bc):
    def body(h_r, wc_r, bc_r, oo):
        oo[...] = jax.nn.sigmoid(
            jnp.dot(h_r[...], wc_r[...], preferred_element_type=jnp.float32) + bc_r[...])

    return pl.pallas_call(
        body,
        out_shape=jax.ShapeDtypeStruct((_N, 1), jnp.float32),
    )(h, wc, bc)


def kernel(nodeState, edge_index, embed, Wq, bq, Wk, bk, Wv, bv, Wo, bo,
           g1, be1, W1, b1, W2, b2, g2, be2, Wc, bc):
    src = edge_index[0].astype(jnp.int32)
    dst = edge_index[1].astype(jnp.int32)
    ns = nodeState.astype(jnp.int32)

    d_iota = jnp.arange(_D, dtype=jnp.int32) // _DH
    h_iota = jnp.arange(_H, dtype=jnp.int32)
    sel_a = (d_iota[:, None] == h_iota[None, :]).astype(jnp.float32)
    sel_b = sel_a.T.copy()
    sel_p = (h_iota[:, None] == jnp.arange(16, dtype=jnp.int32)[None, :]
             ).astype(jnp.float32)

    npad = 10240
    idx0 = jnp.pad(ns, (0, npad - _N))
    h = _sc_gather_rows(embed, idx0)[:_N]

    for l in range(2):
        q, k_, v = _tc_qkv(h, Wq[l], bq[l].reshape(1, -1), Wk[l],
                           bk[l].reshape(1, -1), Wv[l], bv[l].reshape(1, -1))
        ke, qe, ve = _sc_gather_edges(k_, q, v, src, dst)
        wrows = _tc_edge_weight(ke, qe, ve, sel_a, sel_b, sel_p)
        parts = _sc_scatter_add(wrows, dst)
        h = _tc_post(h, parts, sel_b, Wo[l], bo[l].reshape(1, -1),
                     g1[l].reshape(1, -1), be1[l].reshape(1, -1), W1[l],
                     b1[l].reshape(1, -1), W2[l], b2[l].reshape(1, -1),
                     g2[l].reshape(1, -1), be2[l].reshape(1, -1))

    return _tc_final(h, Wc, bc.reshape(1, 1))
